# baseline (speedup 1.0000x reference)
import functools

import jax
import jax.numpy as jnp
from jax import lax
from jax.experimental import pallas as pl
from jax.experimental.pallas import tpu as pltpu
from jax.experimental.pallas import tpu_sc as plsc

_BATCH = 2048
_N_ANTS = 50
_D = 256
_EPS = 1e-7
_L = 16
_KB = _D // _L
_OUTP = 64
_NA_PAD = 56


@functools.cache
def _build_scorer():
    info = plsc.get_sparse_core_info()
    nc, ns = info.num_cores, info.num_subcores
    nw = nc * ns
    bpw = _BATCH // nw

    def body(x_hbm, idx_hbm, q_hbm, rough_hbm, out_hbm,
             idx_v, q_v, rough_v, g0_v, g1_v, out_v, sem0, sem1):
        wid = lax.axis_index("s") * nc + lax.axis_index("c")
        base = wid * bpw
        pltpu.sync_copy(idx_hbm.at[pl.ds(base, bpw)], idx_v)
        pltpu.sync_copy(q_hbm.at[pl.ds(base, bpw)], q_v)
        pltpu.sync_copy(rough_hbm.at[pl.ds(base, bpw)], rough_v)
        bufs = ((g0_v, sem0), (g1_v, sem1))

        def start_gather(b, buf, sem):
            pltpu.make_async_copy(x_hbm.at[idx_v.at[b]], buf, sem).start()

        def wait_gather(b, buf, sem):
            pltpu.make_async_copy(x_hbm.at[idx_v.at[b]], buf, sem).wait()

        zero16 = jnp.zeros((_L,), jnp.float32)
        iota16 = lax.iota(jnp.int32, _L)
        lane_masks = [iota16 == i for i in range(_L)]
        perms = [iota16 ^ w for w in (8, 4, 2, 1)]
        n_full = _N_ANTS // _L
        n_tail = _N_ANTS - n_full * _L

        himask = jnp.full((_L,), 0xFFFF0000, jnp.uint32)

        def unpack2(ref, row, k):
            u = ref[row, pl.ds(_L * k, _L)]
            ev = plsc.bitcast(u << 16, jnp.float32)
            od = plsc.bitcast(u & himask, jnp.float32)
            return ev, od

        def compute_row(bl, g_v):
            qv = [unpack2(q_v, bl, k) for k in range(_KB // 2)]

            def tile_dots(t, n_in_tile):
                res = zero16
                for i in range(n_in_tile):
                    row = t * _L + i
                    acc = zero16
                    for k in range(_KB // 2):
                        ge, go = unpack2(g_v, row, k)
                        acc = acc + ge * qv[k][0] + go * qv[k][1]
                    for p in perms:
                        acc = acc + acc[p]
                    res = jnp.where(lane_masks[i], acc, res)
                return res

            def t_body(t, c):
                out_v[bl, pl.ds(_L * t, _L)] = (
                    rough_v[bl, pl.ds(_L * t, _L)] + 2.0 * tile_dots(t, _L))
                return c

            lax.fori_loop(0, n_full, t_body, 0)
            out_v[bl, pl.ds(_L * n_full, _L)] = (
                rough_v[bl, pl.ds(_L * n_full, _L)]
                + 2.0 * tile_dots(n_full, n_tail))

        start_gather(0, *bufs[0])

        def pair_body(i, carry):
            for par in (0, 1):
                b = 2 * i + par
                nbuf, nsem = bufs[1 - par]

                @pl.when(b + 1 < bpw)
                def _():
                    start_gather(b + 1, nbuf, nsem)

                buf, sem = bufs[par]
                wait_gather(b, buf, sem)
                compute_row(b, buf)
            return carry

        lax.fori_loop(0, bpw // 2, pair_body, 0)
        pltpu.sync_copy(out_v, out_hbm.at[pl.ds(base, bpw)])

    return pl.kernel(
        body,
        out_type=jax.ShapeDtypeStruct((_BATCH, _OUTP), jnp.float32),
        mesh=plsc.VectorSubcoreMesh(core_axis_name="c", subcore_axis_name="s"),
        compiler_params=pltpu.CompilerParams(needs_layout_passes=False),
        scratch_types=[
            pltpu.VMEM((bpw, _NA_PAD), jnp.int32),
            pltpu.VMEM((bpw, _D // 2), jnp.uint32),
            pltpu.VMEM((bpw, _OUTP), jnp.float32),
            pltpu.VMEM((_NA_PAD, _D // 2), jnp.uint32),
            pltpu.VMEM((_NA_PAD, _D // 2), jnp.uint32),
            pltpu.VMEM((bpw, _OUTP), jnp.float32),
            pltpu.SemaphoreType.DMA,
            pltpu.SemaphoreType.DMA,
        ],
    )


def kernel(all_mentions, mentions_batch, pw_batch, top_indices_batch,
           top_rough_scores_batch, current_i, nominal_batch_size,
           W1, a_src1, a_dst1, We1, a_e1, b1,
           W2, a_src2, a_dst2, We2, a_e2, b2):
    x_bf = all_mentions.astype(jnp.bfloat16)
    x_u32 = lax.bitcast_convert_type(
        x_bf.reshape(all_mentions.shape[0], _D // 2, 2), jnp.uint32)
    q_u32 = lax.dynamic_slice_in_dim(x_u32, current_i, _BATCH)
    rough_p = jnp.pad(top_rough_scores_batch,
                      ((0, 0), (0, _OUTP - _N_ANTS)))
    ti = top_indices_batch.astype(jnp.int32)
    idx_p = jnp.concatenate([ti, ti[:, :_NA_PAD - _N_ANTS]], axis=1)
    scores = _build_scorer()(x_u32, idx_p, q_u32, rough_p)
    eps_col = jnp.full((_BATCH, 1), _EPS, jnp.float32)
    return jnp.concatenate([eps_col, scores[:, :_N_ANTS]], axis=1)


# speedup vs baseline: 1.2997x; 1.2997x over previous
import functools

import jax
import jax.numpy as jnp
from jax import lax
from jax.experimental import pallas as pl
from jax.experimental.pallas import tpu as pltpu
from jax.experimental.pallas import tpu_sc as plsc

_BATCH = 2048
_N_ANTS = 50
_D = 256
_EPS = 1e-7
_L = 16
_KB = _D // _L
_OUTP = 64
_NA_PAD = 56


@functools.cache
def _build_scorer():
    info = plsc.get_sparse_core_info()
    nc, ns = info.num_cores, info.num_subcores
    nw = nc * ns
    bpw = _BATCH // nw

    def body(x_hbm, idx_hbm, q_hbm, rough_hbm, out_hbm,
             idx_v, q_v, rough_v, g0_v, g1_v, out_v, sem0, sem1):
        wid = lax.axis_index("s") * nc + lax.axis_index("c")
        base = wid * bpw
        pltpu.sync_copy(idx_hbm.at[pl.ds(base, bpw)], idx_v)
        pltpu.sync_copy(q_hbm.at[pl.ds(base, bpw)], q_v)
        pltpu.sync_copy(rough_hbm.at[pl.ds(base, bpw)], rough_v)
        bufs = ((g0_v, sem0), (g1_v, sem1))

        def start_gather(b, buf, sem):
            pltpu.make_async_copy(x_hbm.at[idx_v.at[b]], buf, sem).start()

        def wait_gather(b, buf, sem):
            pltpu.make_async_copy(x_hbm.at[idx_v.at[b]], buf, sem).wait()

        zero16 = jnp.zeros((_L,), jnp.float32)
        iota16 = lax.iota(jnp.int32, _L)
        lane_masks = [iota16 == i for i in range(_L)]
        perms = [iota16 ^ w for w in (8, 4, 2, 1)]
        n_full = _N_ANTS // _L
        n_tail = _N_ANTS - n_full * _L

        def compute_row(bl, g_v):
            qv = [q_v[bl, pl.ds(_L * k, _L)] for k in range(_KB)]

            def tile_dots(t, n_in_tile):
                res = zero16
                for i in range(n_in_tile):
                    row = t * _L + i
                    acc = g_v[row, pl.ds(0, _L)] * qv[0]
                    for k in range(1, _KB):
                        acc = acc + g_v[row, pl.ds(_L * k, _L)] * qv[k]
                    for p in perms:
                        acc = acc + acc[p]
                    res = jnp.where(lane_masks[i], acc, res)
                return res

            def t_body(t, c):
                out_v[bl, pl.ds(_L * t, _L)] = (
                    rough_v[bl, pl.ds(_L * t, _L)] + 2.0 * tile_dots(t, _L))
                return c

            lax.fori_loop(0, n_full, t_body, 0)
            out_v[bl, pl.ds(_L * n_full, _L)] = (
                rough_v[bl, pl.ds(_L * n_full, _L)]
                + 2.0 * tile_dots(n_full, n_tail))

        start_gather(0, *bufs[0])

        def pair_body(i, carry):
            for par in (0, 1):
                b = 2 * i + par
                nbuf, nsem = bufs[1 - par]

                @pl.when(b + 1 < bpw)
                def _():
                    start_gather(b + 1, nbuf, nsem)

                buf, sem = bufs[par]
                wait_gather(b, buf, sem)
                compute_row(b, buf)
            return carry

        lax.fori_loop(0, bpw // 2, pair_body, 0)
        pltpu.sync_copy(out_v, out_hbm.at[pl.ds(base, bpw)])

    return pl.kernel(
        body,
        out_type=jax.ShapeDtypeStruct((_BATCH, _OUTP), jnp.float32),
        mesh=plsc.VectorSubcoreMesh(core_axis_name="c", subcore_axis_name="s"),
        compiler_params=pltpu.CompilerParams(needs_layout_passes=False),
        scratch_types=[
            pltpu.VMEM((bpw, _NA_PAD), jnp.int32),
            pltpu.VMEM((bpw, _D), jnp.float32),
            pltpu.VMEM((bpw, _OUTP), jnp.float32),
            pltpu.VMEM((_NA_PAD, _D), jnp.float32),
            pltpu.VMEM((_NA_PAD, _D), jnp.float32),
            pltpu.VMEM((bpw, _OUTP), jnp.float32),
            pltpu.SemaphoreType.DMA,
            pltpu.SemaphoreType.DMA,
        ],
    )


def kernel(all_mentions, mentions_batch, pw_batch, top_indices_batch,
           top_rough_scores_batch, current_i, nominal_batch_size,
           W1, a_src1, a_dst1, We1, a_e1, b1,
           W2, a_src2, a_dst2, We2, a_e2, b2):
    q = lax.dynamic_slice_in_dim(all_mentions, current_i, _BATCH)
    rough_p = jnp.pad(top_rough_scores_batch,
                      ((0, 0), (0, _OUTP - _N_ANTS)))
    ti = top_indices_batch.astype(jnp.int32)
    idx_p = jnp.concatenate([ti, ti[:, :_NA_PAD - _N_ANTS]], axis=1)
    scores = _build_scorer()(all_mentions, idx_p, q, rough_p)
    eps_col = jnp.full((_BATCH, 1), _EPS, jnp.float32)
    return jnp.concatenate([eps_col, scores[:, :_N_ANTS]], axis=1)
